# baseline (speedup 1.0000x reference)
import jax
import jax.numpy as jnp
from jax import lax
from jax.experimental import pallas as pl
from jax.experimental.pallas import tpu as pltpu
from jax.experimental.pallas import tpu_sc as plsc

B, S, D = 4096, 200, 64
NC, NS = 2, 16
NW = NC * NS
BPW = B // NW
NBUF = 4
LANES = 16


def _gather_body(idxT_hbm, table_hbm, out_hbm, idx_v, rows_v, trans_v,
                 sem_g0, sem_g1, sem_g2, sem_g3, sem_g4,
                 sem_w0, sem_w1, sem_w2, sem_w3, sem_w4):
    sem_g = [sem_g0, sem_g1, sem_g2, sem_g3, sem_g4]
    sem_w = [sem_w0, sem_w1, sem_w2, sem_w3, sem_w4]
    wid = lax.axis_index("s") * NC + lax.axis_index("c")
    b0 = wid * BPW

    pltpu.sync_copy(idxT_hbm.at[:, pl.ds(b0, BPW)], idx_v)


    def gather(s, buf):
        pltpu.async_copy(table_hbm.at[idx_v.at[s]],
                         rows_v.at[buf], sem_g[buf])

    def gather_wait(s, buf):
        pltpu.make_async_copy(table_hbm.at[idx_v.at[s]],
                              rows_v.at[buf], sem_g[buf]).wait()

    def writeback(s, buf):
        pltpu.async_copy(trans_v.at[buf, :, :, pl.ds(0, BPW)],
                         out_hbm.at[s, :, wid], sem_w[buf])

    def writeback_wait(s, buf):
        pltpu.make_async_copy(trans_v.at[buf, :, :, pl.ds(0, BPW)],
                              out_hbm.at[s, :, wid], sem_w[buf]).wait()

    lane = lax.iota(jnp.int32, LANES)
    dtv, div, colv = [], [], []
    for dc in range(D // LANES):
        d = lane + dc * LANES
        dtv.append(d // 8)
        div.append(d % 8)
        colv.append(d)

    def transpose(buf):
        rows = rows_v.at[buf]
        trans = trans_v.at[buf]

        def chunk(bb, _):
            for j in range(0, LANES, 8):
                vs = []
                for k in range(8):
                    b = bb * LANES + j + k
                    bv = jnp.full((LANES,), b, jnp.int32)
                    for dc in range(D // LANES):
                        vs.append((dc, bv,
                                   rows[b, pl.ds(dc * LANES, LANES)]))
                for dc, bv, v in vs:
                    plsc.store_scatter(trans, [dtv[dc], div[dc], bv], v)
            return 0

        lax.fori_loop(0, BPW // LANES, chunk, 0)

    for s in range(NBUF - 1):
        gather(s, s)

    def quad(p, _):
        for bme in range(NBUF):
            s = NBUF * p + bme
            pre = (bme + NBUF - 1) % NBUF

            @pl.when(s + NBUF - 1 < S)
            def _():
                gather(s + NBUF - 1, pre)

            gather_wait(s, bme)

            @pl.when(s >= NBUF)
            def _():
                writeback_wait(s - NBUF, bme)

            transpose(bme)
            writeback(s, bme)
        return 0

    lax.fori_loop(0, S // NBUF, quad, 0)
    for bme in range(NBUF):
        writeback_wait(S - NBUF + bme, bme)


@jax.jit
def kernel(recipe_indices, embedding_weight):
    idx_t = jnp.transpose(recipe_indices.astype(jnp.int32))
    mesh = plsc.VectorSubcoreMesh(
        core_axis_name="c", subcore_axis_name="s",
        num_cores=NC, num_subcores=NS)
    out5 = pl.kernel(
        _gather_body,
        out_type=jax.ShapeDtypeStruct((S, D // 8, NW, 8, BPW), jnp.float32),
        mesh=mesh,
        scratch_types=[
            pltpu.VMEM((S, BPW), jnp.int32),
            pltpu.VMEM((NBUF, BPW, D), jnp.float32),
            pltpu.VMEM((NBUF, D // 8, 8, BPW + 1), jnp.float32),
        ] + [pltpu.SemaphoreType.DMA] * 10,
        compiler_params=pltpu.CompilerParams(use_tc_tiling_on_sc=False,
                                             needs_layout_passes=False,
                                             disable_bounds_checks=True),
    )(idx_t, embedding_weight)
    return jnp.transpose(out5, (2, 4, 0, 1, 3)).reshape(B, S, D)


# speedup vs baseline: 1.0478x; 1.0478x over previous
import jax
import jax.numpy as jnp
from jax import lax
from jax.experimental import pallas as pl
from jax.experimental.pallas import tpu as pltpu
from jax.experimental.pallas import tpu_sc as plsc

B, S, D = 4096, 200, 64
NC, NS = 2, 16
NW = NC * NS
BPW = B // NW
NBUF = 4
LANES = 16


def _gather_body(idxT_hbm, table_hbm, out_hbm, idx_v, rows_v, trans_v,
                 sem_g0, sem_g1, sem_g2, sem_g3, sem_g4,
                 sem_w0, sem_w1, sem_w2, sem_w3, sem_w4):
    sem_g = [sem_g0, sem_g1, sem_g2, sem_g3, sem_g4]
    sem_w = [sem_w0, sem_w1, sem_w2, sem_w3, sem_w4]
    wid = lax.axis_index("s") * NC + lax.axis_index("c")
    b0 = wid * BPW

    pltpu.sync_copy(idxT_hbm.at[:, pl.ds(b0, BPW)], idx_v)


    def gather(s, buf):
        pltpu.async_copy(table_hbm.at[idx_v.at[s]],
                         rows_v.at[buf], sem_g[buf])

    def gather_wait(s, buf):
        pltpu.make_async_copy(table_hbm.at[idx_v.at[s]],
                              rows_v.at[buf], sem_g[buf]).wait()

    def writeback(s, buf):
        pltpu.async_copy(trans_v.at[buf, :, :, pl.ds(0, BPW)],
                         out_hbm.at[s, :, wid], sem_w[buf])

    def writeback_wait(s, buf):
        pltpu.make_async_copy(trans_v.at[buf, :, :, pl.ds(0, BPW)],
                              out_hbm.at[s, :, wid], sem_w[buf]).wait()

    lane = lax.iota(jnp.int32, LANES)
    dtv, div, colv = [], [], []
    for dc in range(D // LANES):
        d = lane + dc * LANES
        dtv.append(d // 8)
        div.append(d % 8)
        colv.append(d)

    def transpose(buf):
        rows = rows_v.at[buf]
        trans = trans_v.at[buf]

        def chunk(bb, _):
            for j in range(0, LANES, 4):
                vs = []
                for k in range(4):
                    b = bb * LANES + j + k
                    bv = jnp.full((LANES,), b, jnp.int32)
                    for dc in range(D // LANES):
                        vs.append((dc, bv,
                                   rows[b, pl.ds(dc * LANES, LANES)]))
                for dc, bv, v in vs:
                    plsc.store_scatter(trans, [dtv[dc], div[dc], bv], v)
            return 0

        lax.fori_loop(0, BPW // LANES, chunk, 0)

    for s in range(NBUF - 1):
        gather(s, s)

    def quad(p, _):
        for bme in range(NBUF):
            s = NBUF * p + bme
            pre = (bme + NBUF - 1) % NBUF

            @pl.when(s + NBUF - 1 < S)
            def _():
                gather(s + NBUF - 1, pre)

            gather_wait(s, bme)

            @pl.when(s >= NBUF)
            def _():
                writeback_wait(s - NBUF, bme)

            transpose(bme)
            writeback(s, bme)
        return 0

    lax.fori_loop(0, S // NBUF, quad, 0)
    for bme in range(NBUF):
        writeback_wait(S - NBUF + bme, bme)


@jax.jit
def kernel(recipe_indices, embedding_weight):
    idx_t = jnp.transpose(recipe_indices.astype(jnp.int32))
    mesh = plsc.VectorSubcoreMesh(
        core_axis_name="c", subcore_axis_name="s",
        num_cores=NC, num_subcores=NS)
    out5 = pl.kernel(
        _gather_body,
        out_type=jax.ShapeDtypeStruct((S, D // 8, NW, 8, BPW), jnp.float32),
        mesh=mesh,
        scratch_types=[
            pltpu.VMEM((S, BPW), jnp.int32),
            pltpu.VMEM((NBUF, BPW, D), jnp.float32),
            pltpu.VMEM((NBUF, D // 8, 8, BPW + 1), jnp.float32),
        ] + [pltpu.SemaphoreType.DMA] * 10,
        compiler_params=pltpu.CompilerParams(use_tc_tiling_on_sc=False,
                                             needs_layout_passes=False,
                                             disable_bounds_checks=True),
    )(idx_t, embedding_weight)
    return jnp.transpose(out5, (2, 4, 0, 1, 3)).reshape(B, S, D)
